# baseline (device time: 608731 ns/iter reference)
import jax
import jax.numpy as jnp
from jax import lax
from jax.experimental import pallas as pl
from jax.experimental.pallas import tpu as pltpu

T = 1024
D = 2048
V_HALF = 16384
V = 2 * V_HALF
TILE = 512
N_TILES = V_HALF // TILE
PAIRS = N_TILES // 2
K_H = 10
D_N = PAIRS - K_H
SLANES = 128

_FWD_STEP = [10, 13, 16, 18, 20, 22, 25, 27, 29, 31][:K_H]


def _gemm_headsend(x, W):

    def body(x_ref, w_ref, e_ref, zinv_ref, nbr_raw_ref, nbr_raw_x_ref,
             canvas_ref, s_ref, head, stat_buf, stat_recv,
             hd_send_sems, hd_recv_sems, fwdr_send_sems, fwdr_recv_sems,
             stat_send_sem, stat_recv_sem):
        j = pl.program_id(0)
        my_x = lax.axis_index("x")
        my_y = lax.axis_index("y")
        ynbr = (my_x, 1 - my_y)
        xnbr = (1 - my_x, my_y)

        e = jnp.exp(jnp.dot(x_ref[...], w_ref[...],
                            preferred_element_type=jnp.float32))
        e_ref[...] = e
        s_t = jnp.sum(e, axis=1, keepdims=True)

        @pl.when(j == 0)
        def _():
            s_ref[...] = jnp.broadcast_to(s_t, (T, SLANES))

        @pl.when(j > 0)
        def _():
            s_ref[...] = s_ref[...] + jnp.broadcast_to(s_t, (T, SLANES))

        def head_rdma(k):
            return pltpu.make_async_remote_copy(
                src_ref=head.at[k],
                dst_ref=nbr_raw_ref.at[:, pl.ds(k * TILE, TILE)],
                send_sem=hd_send_sems.at[k],
                recv_sem=hd_recv_sems.at[k],
                device_id=ynbr,
                device_id_type=pl.DeviceIdType.MESH,
            )

        def fwd_rdma(k):
            return pltpu.make_async_remote_copy(
                src_ref=nbr_raw_ref.at[:, pl.ds(k * TILE, TILE)],
                dst_ref=nbr_raw_x_ref.at[:, pl.ds(k * TILE, TILE)],
                send_sem=fwdr_send_sems.at[k],
                recv_sem=fwdr_recv_sems.at[k],
                device_id=xnbr,
                device_id_type=pl.DeviceIdType.MESH,
            )

        for k in range(K_H):
            @pl.when(j == 2 * k + my_x)
            def _(k=k):
                head[k, :, :] = e
                head_rdma(k).start()

        for k in range(K_H):
            @pl.when(j == _FWD_STEP[k])
            def _(k=k):
                head_rdma(k).wait_recv()
                fwd_rdma(k).start()

        @pl.when(j == N_TILES - 1)
        def _():
            for k in range(K_H):
                head_rdma(k).wait_send()
                fwd_rdma(k).wait_send()
                fwd_rdma(k).wait_recv()
            stat_buf[...] = s_ref[...]
            rs = pltpu.make_async_remote_copy(
                src_ref=stat_buf,
                dst_ref=stat_recv,
                send_sem=stat_send_sem,
                recv_sem=stat_recv_sem,
                device_id=ynbr,
                device_id_type=pl.DeviceIdType.MESH,
            )
            rs.start()
            rs.wait()
            z = s_ref[:, :1] + stat_recv[:, :1]
            zinv_ref[...] = jnp.broadcast_to(1.0 / z, (T, SLANES))

    return pl.pallas_call(
        body,
        grid=(N_TILES,),
        in_specs=[
            pl.BlockSpec((T, D), lambda j: (0, 0)),
            pl.BlockSpec((D, TILE), lambda j: (0, j)),
        ],
        out_specs=[
            pl.BlockSpec((T, TILE), lambda j: (0, j)),
            pl.BlockSpec((T, SLANES), lambda j: (0, 0)),
            pl.BlockSpec(memory_space=pl.ANY),
            pl.BlockSpec(memory_space=pl.ANY),
            pl.BlockSpec(memory_space=pl.ANY),
        ],
        out_shape=[
            jax.ShapeDtypeStruct((T, V_HALF), jnp.float32),
            jax.ShapeDtypeStruct((T, SLANES), jnp.float32),
            jax.ShapeDtypeStruct((T, K_H * TILE), jnp.float32),
            jax.ShapeDtypeStruct((T, K_H * TILE), jnp.float32),
            jax.ShapeDtypeStruct((T, V), jnp.float32),
        ],
        scratch_shapes=[
            pltpu.VMEM((T, SLANES), jnp.float32),
            pltpu.VMEM((K_H, T, TILE), jnp.float32),
            pltpu.VMEM((T, SLANES), jnp.float32),
            pltpu.VMEM((T, SLANES), jnp.float32),
            pltpu.SemaphoreType.DMA((K_H,)),
            pltpu.SemaphoreType.DMA((K_H,)),
            pltpu.SemaphoreType.DMA((K_H,)),
            pltpu.SemaphoreType.DMA((K_H,)),
            pltpu.SemaphoreType.DMA,
            pltpu.SemaphoreType.DMA,
        ],
        compiler_params=pltpu.CompilerParams(
            has_side_effects=True, vmem_limit_bytes=100 * 1024 * 1024),
    )(x, W)


def _normalize_exchange(e_arr, zinv, nbr_raw, nbr_raw_x, canvas):

    def body(e_ref, zinv_ref, nbr_raw_ref, nbr_raw_x_ref, canvas_ref, out_ref,
             snd, dsnd, rawy_t, rawx_t, cp_sems, dsn_send_sems, d_recv_sems,
             fwd_send_sems, fwd_recv_sems, rawy_ld, rawy_st, rawx_ld,
             rawx_st):
        j = pl.program_id(0)
        kp = lax.rem(j + K_H, PAIRS)
        slot = lax.rem(j, 2)
        my_x = lax.axis_index("x")
        my_y = lax.axis_index("y")
        ynbr = (my_x, 1 - my_y)
        xnbr = (1 - my_x, my_y)
        my_col = my_y * V_HALF + kp * 2 * TILE
        dcol = my_col + my_x * TILE

        def pair_cp(sl, col):
            return pltpu.make_async_copy(
                snd.at[sl], out_ref.at[:, pl.ds(col, 2 * TILE)],
                cp_sems.at[sl])

        def direct_send(k):
            col = my_y * V_HALF + (k + K_H) * 2 * TILE + my_x * TILE
            return pltpu.make_async_remote_copy(
                src_ref=dsnd.at[k],
                dst_ref=out_ref.at[:, pl.ds(col, TILE)],
                send_sem=dsn_send_sems.at[k],
                recv_sem=d_recv_sems.at[k + K_H],
                device_id=ynbr,
                device_id_type=pl.DeviceIdType.MESH,
            )

        def fwd_rdma(pair_idx):
            fcol = (1 - my_y) * V_HALF + pair_idx * 2 * TILE + my_x * TILE
            return pltpu.make_async_remote_copy(
                src_ref=out_ref.at[:, pl.ds(fcol, TILE)],
                dst_ref=out_ref.at[:, pl.ds(fcol, TILE)],
                send_sem=fwd_send_sems.at[pair_idx],
                recv_sem=fwd_recv_sems.at[pair_idx],
                device_id=xnbr,
                device_id_type=pl.DeviceIdType.MESH,
            )

        @pl.when(j >= 2)
        def _():
            kp2 = lax.rem(j - 2 + K_H, PAIRS)
            col2 = my_y * V_HALF + kp2 * 2 * TILE
            pair_cp(slot, col2).wait()

        val = e_ref[...] * zinv_ref[:, :1]
        snd[slot, :, :] = val
        pair_cp(slot, my_col).start()

        for k in range(D_N):
            @pl.when(j == k)
            def _(k=k):
                dsnd[k, :, :] = jnp.where(
                    my_x == 0, val[:, :TILE], val[:, TILE:])
                direct_send(k).start()

        @pl.when(j >= D_N)
        def _():
            ycol = (1 - my_y) * V_HALF + (2 * kp + my_x) * TILE
            ld = pltpu.make_async_copy(
                nbr_raw_ref.at[:, pl.ds(kp * TILE, TILE)], rawy_t, rawy_ld)
            ld.start()
            ld.wait()
            rawy_t[...] = rawy_t[...] * zinv_ref[:, :1]
            st = pltpu.make_async_copy(
                rawy_t, out_ref.at[:, pl.ds(ycol, TILE)], rawy_st)
            st.start()
            xcol = (1 - my_y) * V_HALF + (2 * kp + 1 - my_x) * TILE
            ld2 = pltpu.make_async_copy(
                nbr_raw_x_ref.at[:, pl.ds(kp * TILE, TILE)], rawx_t, rawx_ld)
            ld2.start()
            ld2.wait()
            rawx_t[...] = rawx_t[...] * zinv_ref[:, :1]
            st2 = pltpu.make_async_copy(
                rawx_t, out_ref.at[:, pl.ds(xcol, TILE)], rawx_st)
            st2.start()
            st.wait()
            st2.wait()

        @pl.when(j == PAIRS - 1)
        def _():
            for k in range(K_H, PAIRS):
                fcol = (1 - my_y) * V_HALF + k * 2 * TILE + my_x * TILE
                arr = pltpu.make_async_remote_copy(
                    src_ref=snd.at[0, :, pl.ds(0, TILE)],
                    dst_ref=out_ref.at[:, pl.ds(fcol, TILE)],
                    send_sem=dsn_send_sems.at[0],
                    recv_sem=d_recv_sems.at[k],
                    device_id=ynbr,
                    device_id_type=pl.DeviceIdType.MESH,
                )
                arr.wait_recv()
                fwd_rdma(k).start()
            for dj in (PAIRS - 2, PAIRS - 1):
                sl = dj % 2
                kpd = (dj + K_H) % PAIRS
                pair_cp(sl, my_y * V_HALF + kpd * 2 * TILE).wait()
            for k in range(D_N):
                direct_send(k).wait_send()
            for k in range(K_H, PAIRS):
                fwd_rdma(k).wait_send()
                fcol_in = ((1 - my_y) * V_HALF + k * 2 * TILE
                           + (1 - my_x) * TILE)
                arr = pltpu.make_async_remote_copy(
                    src_ref=snd.at[0, :, pl.ds(0, TILE)],
                    dst_ref=out_ref.at[:, pl.ds(fcol_in, TILE)],
                    send_sem=dsn_send_sems.at[0],
                    recv_sem=fwd_recv_sems.at[k],
                    device_id=xnbr,
                    device_id_type=pl.DeviceIdType.MESH,
                )
                arr.wait_recv()

    return pl.pallas_call(
        body,
        grid=(PAIRS,),
        in_specs=[
            pl.BlockSpec((T, 2 * TILE), lambda j: (0, (j + K_H) % PAIRS)),
            pl.BlockSpec((T, SLANES), lambda j: (0, 0)),
            pl.BlockSpec(memory_space=pl.ANY),
            pl.BlockSpec(memory_space=pl.ANY),
            pl.BlockSpec(memory_space=pl.ANY),
        ],
        out_specs=pl.BlockSpec(memory_space=pl.ANY),
        out_shape=jax.ShapeDtypeStruct((T, V), jnp.float32),
        input_output_aliases={4: 0},
        scratch_shapes=[
            pltpu.VMEM((2, T, 2 * TILE), jnp.float32),
            pltpu.VMEM((D_N, T, TILE), jnp.float32),
            pltpu.VMEM((T, TILE), jnp.float32),
            pltpu.VMEM((T, TILE), jnp.float32),
            pltpu.SemaphoreType.DMA((2,)),
            pltpu.SemaphoreType.DMA((D_N,)),
            pltpu.SemaphoreType.DMA((PAIRS,)),
            pltpu.SemaphoreType.DMA((PAIRS,)),
            pltpu.SemaphoreType.DMA((PAIRS,)),
            pltpu.SemaphoreType.DMA,
            pltpu.SemaphoreType.DMA,
            pltpu.SemaphoreType.DMA,
            pltpu.SemaphoreType.DMA,
        ],
        compiler_params=pltpu.CompilerParams(
            has_side_effects=True, vmem_limit_bytes=100 * 1024 * 1024),
    )(e_arr, zinv, nbr_raw, nbr_raw_x, canvas)


def kernel(x, W):
    e_arr, zinv, nbr_raw, nbr_raw_x, canvas = _gemm_headsend(x, W)
    return _normalize_exchange(e_arr, zinv, nbr_raw, nbr_raw_x, canvas)


# device time: 555898 ns/iter; 1.0950x vs baseline; 1.0950x over previous
import jax
import jax.numpy as jnp
from jax import lax
from jax.experimental import pallas as pl
from jax.experimental.pallas import tpu as pltpu

T = 1024
D = 2048
V_HALF = 16384
V = 2 * V_HALF
TILE = 512
N_TILES = V_HALF // TILE
PAIRS = N_TILES // 2
K_H = 7
D_N = PAIRS - K_H
SLANES = 128

_FWD_STEP = [10, 13, 16, 18, 20, 22, 25, 27, 29, 31][:K_H]


def _gemm_headsend(x, W):

    def body(x_ref, w_ref, e_ref, zinv_ref, nbr_raw_ref, nbr_raw_x_ref,
             canvas_ref, s_ref, head, stat_buf, stat_recv,
             hd_send_sems, hd_recv_sems, fwdr_send_sems, fwdr_recv_sems,
             stat_send_sem, stat_recv_sem):
        j = pl.program_id(0)
        my_x = lax.axis_index("x")
        my_y = lax.axis_index("y")
        ynbr = (my_x, 1 - my_y)
        xnbr = (1 - my_x, my_y)

        e = jnp.exp(jnp.dot(x_ref[...], w_ref[...],
                            preferred_element_type=jnp.float32))
        e_ref[...] = e
        s_t = jnp.sum(e, axis=1, keepdims=True)

        @pl.when(j == 0)
        def _():
            s_ref[...] = jnp.broadcast_to(s_t, (T, SLANES))

        @pl.when(j > 0)
        def _():
            s_ref[...] = s_ref[...] + jnp.broadcast_to(s_t, (T, SLANES))

        def head_rdma(k):
            return pltpu.make_async_remote_copy(
                src_ref=head.at[k],
                dst_ref=nbr_raw_ref.at[:, pl.ds(k * TILE, TILE)],
                send_sem=hd_send_sems.at[k],
                recv_sem=hd_recv_sems.at[k],
                device_id=ynbr,
                device_id_type=pl.DeviceIdType.MESH,
            )

        def fwd_rdma(k):
            return pltpu.make_async_remote_copy(
                src_ref=nbr_raw_ref.at[:, pl.ds(k * TILE, TILE)],
                dst_ref=nbr_raw_x_ref.at[:, pl.ds(k * TILE, TILE)],
                send_sem=fwdr_send_sems.at[k],
                recv_sem=fwdr_recv_sems.at[k],
                device_id=xnbr,
                device_id_type=pl.DeviceIdType.MESH,
            )

        for k in range(K_H):
            @pl.when(j == 2 * k + my_x)
            def _(k=k):
                head[k, :, :] = e
                head_rdma(k).start()

        for k in range(K_H):
            @pl.when(j == _FWD_STEP[k])
            def _(k=k):
                head_rdma(k).wait_recv()
                fwd_rdma(k).start()

        @pl.when(j == N_TILES - 1)
        def _():
            for k in range(K_H):
                head_rdma(k).wait_send()
                fwd_rdma(k).wait_send()
                fwd_rdma(k).wait_recv()
            stat_buf[...] = s_ref[...]
            rs = pltpu.make_async_remote_copy(
                src_ref=stat_buf,
                dst_ref=stat_recv,
                send_sem=stat_send_sem,
                recv_sem=stat_recv_sem,
                device_id=ynbr,
                device_id_type=pl.DeviceIdType.MESH,
            )
            rs.start()
            rs.wait()
            z = s_ref[:, :1] + stat_recv[:, :1]
            zinv_ref[...] = jnp.broadcast_to(1.0 / z, (T, SLANES))

    return pl.pallas_call(
        body,
        grid=(N_TILES,),
        in_specs=[
            pl.BlockSpec((T, D), lambda j: (0, 0)),
            pl.BlockSpec((D, TILE), lambda j: (0, j)),
        ],
        out_specs=[
            pl.BlockSpec((T, TILE), lambda j: (0, j)),
            pl.BlockSpec((T, SLANES), lambda j: (0, 0)),
            pl.BlockSpec(memory_space=pl.ANY),
            pl.BlockSpec(memory_space=pl.ANY),
            pl.BlockSpec(memory_space=pl.ANY),
        ],
        out_shape=[
            jax.ShapeDtypeStruct((T, V_HALF), jnp.float32),
            jax.ShapeDtypeStruct((T, SLANES), jnp.float32),
            jax.ShapeDtypeStruct((T, K_H * TILE), jnp.float32),
            jax.ShapeDtypeStruct((T, K_H * TILE), jnp.float32),
            jax.ShapeDtypeStruct((T, V), jnp.float32),
        ],
        scratch_shapes=[
            pltpu.VMEM((T, SLANES), jnp.float32),
            pltpu.VMEM((K_H, T, TILE), jnp.float32),
            pltpu.VMEM((T, SLANES), jnp.float32),
            pltpu.VMEM((T, SLANES), jnp.float32),
            pltpu.SemaphoreType.DMA((K_H,)),
            pltpu.SemaphoreType.DMA((K_H,)),
            pltpu.SemaphoreType.DMA((K_H,)),
            pltpu.SemaphoreType.DMA((K_H,)),
            pltpu.SemaphoreType.DMA,
            pltpu.SemaphoreType.DMA,
        ],
        compiler_params=pltpu.CompilerParams(
            has_side_effects=True, vmem_limit_bytes=100 * 1024 * 1024),
    )(x, W)


def _normalize_exchange(e_arr, zinv, nbr_raw, nbr_raw_x, canvas):

    def body(e_ref, zinv_ref, nbr_raw_ref, nbr_raw_x_ref, canvas_ref, out_ref,
             snd, rawy_t, rawx_t, cp_sems, snd_send_sems, d_recv_sems,
             fwd_send_sems, fwd_recv_sems, rawy_ld, rawy_st, rawx_ld,
             rawx_st):
        j = pl.program_id(0)
        kp = lax.rem(j + K_H, PAIRS)
        slot = lax.rem(j, 2)
        my_x = lax.axis_index("x")
        my_y = lax.axis_index("y")
        ynbr = (my_x, 1 - my_y)
        xnbr = (1 - my_x, my_y)
        my_col = my_y * V_HALF + kp * 2 * TILE
        dcol = my_col + my_x * TILE

        def pair_cp(sl, col):
            return pltpu.make_async_copy(
                snd.at[sl], out_ref.at[:, pl.ds(col, 2 * TILE)],
                cp_sems.at[sl])

        def direct_send(sl, col, pair_idx):
            return pltpu.make_async_remote_copy(
                src_ref=snd.at[sl, :, pl.ds(my_x * TILE, TILE)],
                dst_ref=out_ref.at[:, pl.ds(col, TILE)],
                send_sem=snd_send_sems.at[sl],
                recv_sem=d_recv_sems.at[pair_idx],
                device_id=ynbr,
                device_id_type=pl.DeviceIdType.MESH,
            )

        def fwd_rdma(pair_idx):
            fcol = (1 - my_y) * V_HALF + pair_idx * 2 * TILE + my_x * TILE
            return pltpu.make_async_remote_copy(
                src_ref=out_ref.at[:, pl.ds(fcol, TILE)],
                dst_ref=out_ref.at[:, pl.ds(fcol, TILE)],
                send_sem=fwd_send_sems.at[pair_idx],
                recv_sem=fwd_recv_sems.at[pair_idx],
                device_id=xnbr,
                device_id_type=pl.DeviceIdType.MESH,
            )

        @pl.when(j >= 2)
        def _():
            kp2 = lax.rem(j - 2 + K_H, PAIRS)
            col2 = my_y * V_HALF + kp2 * 2 * TILE
            pair_cp(slot, col2).wait()

            @pl.when(j - 2 < D_N)
            def _():
                direct_send(slot, col2 + my_x * TILE, kp2).wait_send()

        snd[slot, :, :] = e_ref[...] * zinv_ref[:, :1]
        pair_cp(slot, my_col).start()

        @pl.when(j < D_N)
        def _():
            direct_send(slot, dcol, kp).start()

        @pl.when((j >= 2) & (j < 2 + D_N))
        def _():
            kf = j - 2 + K_H
            fcol = (1 - my_y) * V_HALF + kf * 2 * TILE + my_x * TILE
            arr = pltpu.make_async_remote_copy(
                src_ref=snd.at[0, :, pl.ds(0, TILE)],
                dst_ref=out_ref.at[:, pl.ds(fcol, TILE)],
                send_sem=snd_send_sems.at[0],
                recv_sem=d_recv_sems.at[kf],
                device_id=ynbr,
                device_id_type=pl.DeviceIdType.MESH,
            )
            arr.wait_recv()
            fwd_rdma(kf).start()

        @pl.when(j >= D_N)
        def _():
            ycol = (1 - my_y) * V_HALF + (2 * kp + my_x) * TILE
            ld = pltpu.make_async_copy(
                nbr_raw_ref.at[:, pl.ds(kp * TILE, TILE)], rawy_t, rawy_ld)
            ld.start()
            ld.wait()
            rawy_t[...] = rawy_t[...] * zinv_ref[:, :1]
            st = pltpu.make_async_copy(
                rawy_t, out_ref.at[:, pl.ds(ycol, TILE)], rawy_st)
            st.start()
            xcol = (1 - my_y) * V_HALF + (2 * kp + 1 - my_x) * TILE
            ld2 = pltpu.make_async_copy(
                nbr_raw_x_ref.at[:, pl.ds(kp * TILE, TILE)], rawx_t, rawx_ld)
            ld2.start()
            ld2.wait()
            rawx_t[...] = rawx_t[...] * zinv_ref[:, :1]
            st2 = pltpu.make_async_copy(
                rawx_t, out_ref.at[:, pl.ds(xcol, TILE)], rawx_st)
            st2.start()
            st.wait()
            st2.wait()

        @pl.when(j == PAIRS - 1)
        def _():
            for dj in (PAIRS - 2, PAIRS - 1):
                sl = dj % 2
                kpd = (dj + K_H) % PAIRS
                pair_cp(sl, my_y * V_HALF + kpd * 2 * TILE).wait()
            for k in range(K_H, PAIRS):
                fwd_rdma(k).wait_send()
                fcol_in = ((1 - my_y) * V_HALF + k * 2 * TILE
                           + (1 - my_x) * TILE)
                arr = pltpu.make_async_remote_copy(
                    src_ref=snd.at[0, :, pl.ds(0, TILE)],
                    dst_ref=out_ref.at[:, pl.ds(fcol_in, TILE)],
                    send_sem=snd_send_sems.at[0],
                    recv_sem=fwd_recv_sems.at[k],
                    device_id=xnbr,
                    device_id_type=pl.DeviceIdType.MESH,
                )
                arr.wait_recv()

    return pl.pallas_call(
        body,
        grid=(PAIRS,),
        in_specs=[
            pl.BlockSpec((T, 2 * TILE), lambda j: (0, (j + K_H) % PAIRS)),
            pl.BlockSpec((T, SLANES), lambda j: (0, 0)),
            pl.BlockSpec(memory_space=pl.ANY),
            pl.BlockSpec(memory_space=pl.ANY),
            pl.BlockSpec(memory_space=pl.ANY),
        ],
        out_specs=pl.BlockSpec(memory_space=pl.ANY),
        out_shape=jax.ShapeDtypeStruct((T, V), jnp.float32),
        input_output_aliases={4: 0},
        scratch_shapes=[
            pltpu.VMEM((2, T, 2 * TILE), jnp.float32),
            pltpu.VMEM((T, TILE), jnp.float32),
            pltpu.VMEM((T, TILE), jnp.float32),
            pltpu.SemaphoreType.DMA((2,)),
            pltpu.SemaphoreType.DMA((2,)),
            pltpu.SemaphoreType.DMA((PAIRS,)),
            pltpu.SemaphoreType.DMA((PAIRS,)),
            pltpu.SemaphoreType.DMA((PAIRS,)),
            pltpu.SemaphoreType.DMA,
            pltpu.SemaphoreType.DMA,
            pltpu.SemaphoreType.DMA,
            pltpu.SemaphoreType.DMA,
        ],
        compiler_params=pltpu.CompilerParams(
            has_side_effects=True, vmem_limit_bytes=100 * 1024 * 1024),
    )(e_arr, zinv, nbr_raw, nbr_raw_x, canvas)


def kernel(x, W):
    e_arr, zinv, nbr_raw, nbr_raw_x, canvas = _gemm_headsend(x, W)
    return _normalize_exchange(e_arr, zinv, nbr_raw, nbr_raw_x, canvas)


# device time: 546672 ns/iter; 1.1135x vs baseline; 1.0169x over previous
import jax
import jax.numpy as jnp
from jax import lax
from jax.experimental import pallas as pl
from jax.experimental.pallas import tpu as pltpu

T = 1024
D = 2048
V_HALF = 16384
V = 2 * V_HALF
TILE = 512
N_TILES = V_HALF // TILE
PAIRS = N_TILES // 2
K_H = 6
D_N = PAIRS - K_H
SLANES = 128

_FWD_STEP = [10, 13, 16, 18, 20, 22, 25, 27, 29, 31][:K_H]


def _gemm_headsend(x, W):

    def body(x_ref, w_ref, e_ref, zinv_ref, nbr_raw_ref, nbr_raw_x_ref,
             canvas_ref, s_ref, head, stat_buf, stat_recv,
             hd_send_sems, hd_recv_sems, fwdr_send_sems, fwdr_recv_sems,
             stat_send_sem, stat_recv_sem):
        j = pl.program_id(0)
        my_x = lax.axis_index("x")
        my_y = lax.axis_index("y")
        ynbr = (my_x, 1 - my_y)
        xnbr = (1 - my_x, my_y)

        e = jnp.exp(jnp.dot(x_ref[...], w_ref[...],
                            preferred_element_type=jnp.float32))
        e_ref[...] = e
        s_t = jnp.sum(e, axis=1, keepdims=True)

        @pl.when(j == 0)
        def _():
            s_ref[...] = jnp.broadcast_to(s_t, (T, SLANES))

        @pl.when(j > 0)
        def _():
            s_ref[...] = s_ref[...] + jnp.broadcast_to(s_t, (T, SLANES))

        def head_rdma(k):
            return pltpu.make_async_remote_copy(
                src_ref=head.at[k],
                dst_ref=nbr_raw_ref.at[:, pl.ds(k * TILE, TILE)],
                send_sem=hd_send_sems.at[k],
                recv_sem=hd_recv_sems.at[k],
                device_id=ynbr,
                device_id_type=pl.DeviceIdType.MESH,
            )

        def fwd_rdma(k):
            return pltpu.make_async_remote_copy(
                src_ref=nbr_raw_ref.at[:, pl.ds(k * TILE, TILE)],
                dst_ref=nbr_raw_x_ref.at[:, pl.ds(k * TILE, TILE)],
                send_sem=fwdr_send_sems.at[k],
                recv_sem=fwdr_recv_sems.at[k],
                device_id=xnbr,
                device_id_type=pl.DeviceIdType.MESH,
            )

        for k in range(K_H):
            @pl.when(j == 2 * k + my_x)
            def _(k=k):
                head[k, :, :] = e
                head_rdma(k).start()

        for k in range(K_H):
            @pl.when(j == _FWD_STEP[k])
            def _(k=k):
                head_rdma(k).wait_recv()
                fwd_rdma(k).start()

        @pl.when(j == N_TILES - 1)
        def _():
            for k in range(K_H):
                head_rdma(k).wait_send()
                fwd_rdma(k).wait_send()
                fwd_rdma(k).wait_recv()
            stat_buf[...] = s_ref[...]
            rs = pltpu.make_async_remote_copy(
                src_ref=stat_buf,
                dst_ref=stat_recv,
                send_sem=stat_send_sem,
                recv_sem=stat_recv_sem,
                device_id=ynbr,
                device_id_type=pl.DeviceIdType.MESH,
            )
            rs.start()
            rs.wait()
            z = s_ref[:, :1] + stat_recv[:, :1]
            zinv_ref[...] = jnp.broadcast_to(1.0 / z, (T, SLANES))

    return pl.pallas_call(
        body,
        grid=(N_TILES,),
        in_specs=[
            pl.BlockSpec((T, D), lambda j: (0, 0)),
            pl.BlockSpec((D, TILE), lambda j: (0, j)),
        ],
        out_specs=[
            pl.BlockSpec((T, TILE), lambda j: (0, j)),
            pl.BlockSpec((T, SLANES), lambda j: (0, 0)),
            pl.BlockSpec(memory_space=pl.ANY),
            pl.BlockSpec(memory_space=pl.ANY),
            pl.BlockSpec(memory_space=pl.ANY),
        ],
        out_shape=[
            jax.ShapeDtypeStruct((T, V_HALF), jnp.float32),
            jax.ShapeDtypeStruct((T, SLANES), jnp.float32),
            jax.ShapeDtypeStruct((T, K_H * TILE), jnp.float32),
            jax.ShapeDtypeStruct((T, K_H * TILE), jnp.float32),
            jax.ShapeDtypeStruct((T, V), jnp.float32),
        ],
        scratch_shapes=[
            pltpu.VMEM((T, SLANES), jnp.float32),
            pltpu.VMEM((K_H, T, TILE), jnp.float32),
            pltpu.VMEM((T, SLANES), jnp.float32),
            pltpu.VMEM((T, SLANES), jnp.float32),
            pltpu.SemaphoreType.DMA((K_H,)),
            pltpu.SemaphoreType.DMA((K_H,)),
            pltpu.SemaphoreType.DMA((K_H,)),
            pltpu.SemaphoreType.DMA((K_H,)),
            pltpu.SemaphoreType.DMA,
            pltpu.SemaphoreType.DMA,
        ],
        compiler_params=pltpu.CompilerParams(
            has_side_effects=True, vmem_limit_bytes=100 * 1024 * 1024),
    )(x, W)


def _normalize_exchange(e_arr, zinv, nbr_raw, nbr_raw_x, canvas):

    def body(e_ref, zinv_ref, nbr_raw_ref, nbr_raw_x_ref, canvas_ref, out_ref,
             snd, rawy_t, rawx_t, cp_sems, snd_send_sems, d_recv_sems,
             fwd_send_sems, fwd_recv_sems, rawy_ld, rawy_st, rawx_ld,
             rawx_st):
        j = pl.program_id(0)
        kp = lax.rem(j + K_H, PAIRS)
        slot = lax.rem(j, 2)
        my_x = lax.axis_index("x")
        my_y = lax.axis_index("y")
        ynbr = (my_x, 1 - my_y)
        xnbr = (1 - my_x, my_y)
        my_col = my_y * V_HALF + kp * 2 * TILE
        dcol = my_col + my_x * TILE

        def pair_cp(sl, col):
            return pltpu.make_async_copy(
                snd.at[sl], out_ref.at[:, pl.ds(col, 2 * TILE)],
                cp_sems.at[sl])

        def direct_send(sl, col, pair_idx):
            return pltpu.make_async_remote_copy(
                src_ref=snd.at[sl, :, pl.ds(my_x * TILE, TILE)],
                dst_ref=out_ref.at[:, pl.ds(col, TILE)],
                send_sem=snd_send_sems.at[sl],
                recv_sem=d_recv_sems.at[pair_idx],
                device_id=ynbr,
                device_id_type=pl.DeviceIdType.MESH,
            )

        def fwd_rdma(pair_idx):
            fcol = (1 - my_y) * V_HALF + pair_idx * 2 * TILE + my_x * TILE
            return pltpu.make_async_remote_copy(
                src_ref=out_ref.at[:, pl.ds(fcol, TILE)],
                dst_ref=out_ref.at[:, pl.ds(fcol, TILE)],
                send_sem=fwd_send_sems.at[pair_idx],
                recv_sem=fwd_recv_sems.at[pair_idx],
                device_id=xnbr,
                device_id_type=pl.DeviceIdType.MESH,
            )

        @pl.when(j >= 2)
        def _():
            kp2 = lax.rem(j - 2 + K_H, PAIRS)
            col2 = my_y * V_HALF + kp2 * 2 * TILE
            pair_cp(slot, col2).wait()

            @pl.when(j - 2 < D_N)
            def _():
                direct_send(slot, col2 + my_x * TILE, kp2).wait_send()

        snd[slot, :, :] = e_ref[...] * zinv_ref[:, :1]
        pair_cp(slot, my_col).start()

        @pl.when(j < D_N)
        def _():
            direct_send(slot, dcol, kp).start()

        @pl.when((j >= 2) & (j < 2 + D_N))
        def _():
            kf = j - 2 + K_H
            fcol = (1 - my_y) * V_HALF + kf * 2 * TILE + my_x * TILE
            arr = pltpu.make_async_remote_copy(
                src_ref=snd.at[0, :, pl.ds(0, TILE)],
                dst_ref=out_ref.at[:, pl.ds(fcol, TILE)],
                send_sem=snd_send_sems.at[0],
                recv_sem=d_recv_sems.at[kf],
                device_id=ynbr,
                device_id_type=pl.DeviceIdType.MESH,
            )
            arr.wait_recv()
            fwd_rdma(kf).start()

        @pl.when(j >= D_N)
        def _():
            ycol = (1 - my_y) * V_HALF + (2 * kp + my_x) * TILE
            ld = pltpu.make_async_copy(
                nbr_raw_ref.at[:, pl.ds(kp * TILE, TILE)], rawy_t, rawy_ld)
            ld.start()
            ld.wait()
            rawy_t[...] = rawy_t[...] * zinv_ref[:, :1]
            st = pltpu.make_async_copy(
                rawy_t, out_ref.at[:, pl.ds(ycol, TILE)], rawy_st)
            st.start()
            xcol = (1 - my_y) * V_HALF + (2 * kp + 1 - my_x) * TILE
            ld2 = pltpu.make_async_copy(
                nbr_raw_x_ref.at[:, pl.ds(kp * TILE, TILE)], rawx_t, rawx_ld)
            ld2.start()
            ld2.wait()
            rawx_t[...] = rawx_t[...] * zinv_ref[:, :1]
            st2 = pltpu.make_async_copy(
                rawx_t, out_ref.at[:, pl.ds(xcol, TILE)], rawx_st)
            st2.start()
            st.wait()
            st2.wait()

        @pl.when(j == PAIRS - 1)
        def _():
            for dj in (PAIRS - 2, PAIRS - 1):
                sl = dj % 2
                kpd = (dj + K_H) % PAIRS
                pair_cp(sl, my_y * V_HALF + kpd * 2 * TILE).wait()
            for k in range(K_H, PAIRS):
                fwd_rdma(k).wait_send()
                fcol_in = ((1 - my_y) * V_HALF + k * 2 * TILE
                           + (1 - my_x) * TILE)
                arr = pltpu.make_async_remote_copy(
                    src_ref=snd.at[0, :, pl.ds(0, TILE)],
                    dst_ref=out_ref.at[:, pl.ds(fcol_in, TILE)],
                    send_sem=snd_send_sems.at[0],
                    recv_sem=fwd_recv_sems.at[k],
                    device_id=xnbr,
                    device_id_type=pl.DeviceIdType.MESH,
                )
                arr.wait_recv()

    return pl.pallas_call(
        body,
        grid=(PAIRS,),
        in_specs=[
            pl.BlockSpec((T, 2 * TILE), lambda j: (0, (j + K_H) % PAIRS)),
            pl.BlockSpec((T, SLANES), lambda j: (0, 0)),
            pl.BlockSpec(memory_space=pl.ANY),
            pl.BlockSpec(memory_space=pl.ANY),
            pl.BlockSpec(memory_space=pl.ANY),
        ],
        out_specs=pl.BlockSpec(memory_space=pl.ANY),
        out_shape=jax.ShapeDtypeStruct((T, V), jnp.float32),
        input_output_aliases={4: 0},
        scratch_shapes=[
            pltpu.VMEM((2, T, 2 * TILE), jnp.float32),
            pltpu.VMEM((T, TILE), jnp.float32),
            pltpu.VMEM((T, TILE), jnp.float32),
            pltpu.SemaphoreType.DMA((2,)),
            pltpu.SemaphoreType.DMA((2,)),
            pltpu.SemaphoreType.DMA((PAIRS,)),
            pltpu.SemaphoreType.DMA((PAIRS,)),
            pltpu.SemaphoreType.DMA((PAIRS,)),
            pltpu.SemaphoreType.DMA,
            pltpu.SemaphoreType.DMA,
            pltpu.SemaphoreType.DMA,
            pltpu.SemaphoreType.DMA,
        ],
        compiler_params=pltpu.CompilerParams(
            has_side_effects=True, vmem_limit_bytes=100 * 1024 * 1024),
    )(e_arr, zinv, nbr_raw, nbr_raw_x, canvas)


def kernel(x, W):
    e_arr, zinv, nbr_raw, nbr_raw_x, canvas = _gemm_headsend(x, W)
    return _normalize_exchange(e_arr, zinv, nbr_raw, nbr_raw_x, canvas)


# device time: 544646 ns/iter; 1.1177x vs baseline; 1.0037x over previous
import jax
import jax.numpy as jnp
from jax import lax
from jax.experimental import pallas as pl
from jax.experimental.pallas import tpu as pltpu

T = 1024
D = 2048
V_HALF = 16384
V = 2 * V_HALF
TILE = 512
N_TILES = V_HALF // TILE
PAIRS = N_TILES // 2
K_H = 4
D_N = PAIRS - K_H
SLANES = 128

_FWD_STEP = [10, 13, 16, 18, 20, 22, 25, 27, 29, 31][:K_H]


def _gemm_headsend(x, W):

    def body(x_ref, w_ref, e_ref, zinv_ref, nbr_raw_ref, nbr_raw_x_ref,
             canvas_ref, s_ref, head, stat_buf, stat_recv,
             hd_send_sems, hd_recv_sems, fwdr_send_sems, fwdr_recv_sems,
             stat_send_sem, stat_recv_sem):
        j = pl.program_id(0)
        my_x = lax.axis_index("x")
        my_y = lax.axis_index("y")
        ynbr = (my_x, 1 - my_y)
        xnbr = (1 - my_x, my_y)

        e = jnp.exp(jnp.dot(x_ref[...], w_ref[...],
                            preferred_element_type=jnp.float32))
        e_ref[...] = e
        s_t = jnp.sum(e, axis=1, keepdims=True)

        @pl.when(j == 0)
        def _():
            s_ref[...] = jnp.broadcast_to(s_t, (T, SLANES))

        @pl.when(j > 0)
        def _():
            s_ref[...] = s_ref[...] + jnp.broadcast_to(s_t, (T, SLANES))

        def head_rdma(k):
            return pltpu.make_async_remote_copy(
                src_ref=head.at[k],
                dst_ref=nbr_raw_ref.at[:, pl.ds(k * TILE, TILE)],
                send_sem=hd_send_sems.at[k],
                recv_sem=hd_recv_sems.at[k],
                device_id=ynbr,
                device_id_type=pl.DeviceIdType.MESH,
            )

        def fwd_rdma(k):
            return pltpu.make_async_remote_copy(
                src_ref=nbr_raw_ref.at[:, pl.ds(k * TILE, TILE)],
                dst_ref=nbr_raw_x_ref.at[:, pl.ds(k * TILE, TILE)],
                send_sem=fwdr_send_sems.at[k],
                recv_sem=fwdr_recv_sems.at[k],
                device_id=xnbr,
                device_id_type=pl.DeviceIdType.MESH,
            )

        for k in range(K_H):
            @pl.when(j == 2 * k + my_x)
            def _(k=k):
                head[k, :, :] = e
                head_rdma(k).start()

        for k in range(K_H):
            @pl.when(j == _FWD_STEP[k])
            def _(k=k):
                head_rdma(k).wait_recv()
                fwd_rdma(k).start()

        @pl.when(j == N_TILES - 1)
        def _():
            for k in range(K_H):
                head_rdma(k).wait_send()
                fwd_rdma(k).wait_send()
                fwd_rdma(k).wait_recv()
            stat_buf[...] = s_ref[...]
            rs = pltpu.make_async_remote_copy(
                src_ref=stat_buf,
                dst_ref=stat_recv,
                send_sem=stat_send_sem,
                recv_sem=stat_recv_sem,
                device_id=ynbr,
                device_id_type=pl.DeviceIdType.MESH,
            )
            rs.start()
            rs.wait()
            z = s_ref[:, :1] + stat_recv[:, :1]
            zinv_ref[...] = jnp.broadcast_to(1.0 / z, (T, SLANES))

    return pl.pallas_call(
        body,
        grid=(N_TILES,),
        in_specs=[
            pl.BlockSpec((T, D), lambda j: (0, 0)),
            pl.BlockSpec((D, TILE), lambda j: (0, j)),
        ],
        out_specs=[
            pl.BlockSpec((T, TILE), lambda j: (0, j)),
            pl.BlockSpec((T, SLANES), lambda j: (0, 0)),
            pl.BlockSpec(memory_space=pl.ANY),
            pl.BlockSpec(memory_space=pl.ANY),
            pl.BlockSpec(memory_space=pl.ANY),
        ],
        out_shape=[
            jax.ShapeDtypeStruct((T, V_HALF), jnp.float32),
            jax.ShapeDtypeStruct((T, SLANES), jnp.float32),
            jax.ShapeDtypeStruct((T, K_H * TILE), jnp.float32),
            jax.ShapeDtypeStruct((T, K_H * TILE), jnp.float32),
            jax.ShapeDtypeStruct((T, V), jnp.float32),
        ],
        scratch_shapes=[
            pltpu.VMEM((T, SLANES), jnp.float32),
            pltpu.VMEM((K_H, T, TILE), jnp.float32),
            pltpu.VMEM((T, SLANES), jnp.float32),
            pltpu.VMEM((T, SLANES), jnp.float32),
            pltpu.SemaphoreType.DMA((K_H,)),
            pltpu.SemaphoreType.DMA((K_H,)),
            pltpu.SemaphoreType.DMA((K_H,)),
            pltpu.SemaphoreType.DMA((K_H,)),
            pltpu.SemaphoreType.DMA,
            pltpu.SemaphoreType.DMA,
        ],
        compiler_params=pltpu.CompilerParams(
            has_side_effects=True, vmem_limit_bytes=100 * 1024 * 1024),
    )(x, W)


def _normalize_exchange(e_arr, zinv, nbr_raw, nbr_raw_x, canvas):

    def body(e_ref, zinv_ref, nbr_raw_ref, nbr_raw_x_ref, canvas_ref, out_ref,
             snd, rawy_t, rawx_t, cp_sems, snd_send_sems, d_recv_sems,
             fwd_send_sems, fwd_recv_sems, rawy_ld, rawy_st, rawx_ld,
             rawx_st):
        j = pl.program_id(0)
        kp = lax.rem(j + K_H, PAIRS)
        slot = lax.rem(j, 2)
        my_x = lax.axis_index("x")
        my_y = lax.axis_index("y")
        ynbr = (my_x, 1 - my_y)
        xnbr = (1 - my_x, my_y)
        my_col = my_y * V_HALF + kp * 2 * TILE
        dcol = my_col + my_x * TILE

        def pair_cp(sl, col):
            return pltpu.make_async_copy(
                snd.at[sl], out_ref.at[:, pl.ds(col, 2 * TILE)],
                cp_sems.at[sl])

        def direct_send(sl, col, pair_idx):
            return pltpu.make_async_remote_copy(
                src_ref=snd.at[sl, :, pl.ds(my_x * TILE, TILE)],
                dst_ref=out_ref.at[:, pl.ds(col, TILE)],
                send_sem=snd_send_sems.at[sl],
                recv_sem=d_recv_sems.at[pair_idx],
                device_id=ynbr,
                device_id_type=pl.DeviceIdType.MESH,
            )

        def fwd_rdma(pair_idx):
            fcol = (1 - my_y) * V_HALF + pair_idx * 2 * TILE + my_x * TILE
            return pltpu.make_async_remote_copy(
                src_ref=out_ref.at[:, pl.ds(fcol, TILE)],
                dst_ref=out_ref.at[:, pl.ds(fcol, TILE)],
                send_sem=fwd_send_sems.at[pair_idx],
                recv_sem=fwd_recv_sems.at[pair_idx],
                device_id=xnbr,
                device_id_type=pl.DeviceIdType.MESH,
            )

        @pl.when(j >= 2)
        def _():
            kp2 = lax.rem(j - 2 + K_H, PAIRS)
            col2 = my_y * V_HALF + kp2 * 2 * TILE
            pair_cp(slot, col2).wait()

            @pl.when(j - 2 < D_N)
            def _():
                direct_send(slot, col2 + my_x * TILE, kp2).wait_send()

        snd[slot, :, :] = e_ref[...] * zinv_ref[:, :1]
        pair_cp(slot, my_col).start()

        @pl.when(j < D_N)
        def _():
            direct_send(slot, dcol, kp).start()

        @pl.when((j >= 2) & (j < 2 + D_N))
        def _():
            kf = j - 2 + K_H
            fcol = (1 - my_y) * V_HALF + kf * 2 * TILE + my_x * TILE
            arr = pltpu.make_async_remote_copy(
                src_ref=snd.at[0, :, pl.ds(0, TILE)],
                dst_ref=out_ref.at[:, pl.ds(fcol, TILE)],
                send_sem=snd_send_sems.at[0],
                recv_sem=d_recv_sems.at[kf],
                device_id=ynbr,
                device_id_type=pl.DeviceIdType.MESH,
            )
            arr.wait_recv()
            fwd_rdma(kf).start()

        @pl.when(j >= D_N)
        def _():
            ycol = (1 - my_y) * V_HALF + (2 * kp + my_x) * TILE
            ld = pltpu.make_async_copy(
                nbr_raw_ref.at[:, pl.ds(kp * TILE, TILE)], rawy_t, rawy_ld)
            ld.start()
            ld.wait()
            rawy_t[...] = rawy_t[...] * zinv_ref[:, :1]
            st = pltpu.make_async_copy(
                rawy_t, out_ref.at[:, pl.ds(ycol, TILE)], rawy_st)
            st.start()
            xcol = (1 - my_y) * V_HALF + (2 * kp + 1 - my_x) * TILE
            ld2 = pltpu.make_async_copy(
                nbr_raw_x_ref.at[:, pl.ds(kp * TILE, TILE)], rawx_t, rawx_ld)
            ld2.start()
            ld2.wait()
            rawx_t[...] = rawx_t[...] * zinv_ref[:, :1]
            st2 = pltpu.make_async_copy(
                rawx_t, out_ref.at[:, pl.ds(xcol, TILE)], rawx_st)
            st2.start()
            st.wait()
            st2.wait()

        @pl.when(j == PAIRS - 1)
        def _():
            for dj in (PAIRS - 2, PAIRS - 1):
                sl = dj % 2
                kpd = (dj + K_H) % PAIRS
                pair_cp(sl, my_y * V_HALF + kpd * 2 * TILE).wait()
            for k in range(K_H, PAIRS):
                fwd_rdma(k).wait_send()
                fcol_in = ((1 - my_y) * V_HALF + k * 2 * TILE
                           + (1 - my_x) * TILE)
                arr = pltpu.make_async_remote_copy(
                    src_ref=snd.at[0, :, pl.ds(0, TILE)],
                    dst_ref=out_ref.at[:, pl.ds(fcol_in, TILE)],
                    send_sem=snd_send_sems.at[0],
                    recv_sem=fwd_recv_sems.at[k],
                    device_id=xnbr,
                    device_id_type=pl.DeviceIdType.MESH,
                )
                arr.wait_recv()

    return pl.pallas_call(
        body,
        grid=(PAIRS,),
        in_specs=[
            pl.BlockSpec((T, 2 * TILE), lambda j: (0, (j + K_H) % PAIRS)),
            pl.BlockSpec((T, SLANES), lambda j: (0, 0)),
            pl.BlockSpec(memory_space=pl.ANY),
            pl.BlockSpec(memory_space=pl.ANY),
            pl.BlockSpec(memory_space=pl.ANY),
        ],
        out_specs=pl.BlockSpec(memory_space=pl.ANY),
        out_shape=jax.ShapeDtypeStruct((T, V), jnp.float32),
        input_output_aliases={4: 0},
        scratch_shapes=[
            pltpu.VMEM((2, T, 2 * TILE), jnp.float32),
            pltpu.VMEM((T, TILE), jnp.float32),
            pltpu.VMEM((T, TILE), jnp.float32),
            pltpu.SemaphoreType.DMA((2,)),
            pltpu.SemaphoreType.DMA((2,)),
            pltpu.SemaphoreType.DMA((PAIRS,)),
            pltpu.SemaphoreType.DMA((PAIRS,)),
            pltpu.SemaphoreType.DMA((PAIRS,)),
            pltpu.SemaphoreType.DMA,
            pltpu.SemaphoreType.DMA,
            pltpu.SemaphoreType.DMA,
            pltpu.SemaphoreType.DMA,
        ],
        compiler_params=pltpu.CompilerParams(
            has_side_effects=True, vmem_limit_bytes=100 * 1024 * 1024),
    )(e_arr, zinv, nbr_raw, nbr_raw_x, canvas)


def kernel(x, W):
    e_arr, zinv, nbr_raw, nbr_raw_x, canvas = _gemm_headsend(x, W)
    return _normalize_exchange(e_arr, zinv, nbr_raw, nbr_raw_x, canvas)
